# baseline (speedup 1.0000x reference)
import jax
import jax.numpy as jnp
from jax.experimental import pallas as pl
from jax.experimental.pallas import tpu as pltpu

T = 4096
D = 2048
E = 8

_BM = 512
_NB = T // _BM
_STEPS = 2 * _NB
_WCHUNKS = 8
_KC = D // _WCHUNKS


def _fused_kernel(pred_ref, wp_ref, bp_ref, x_ref, w_hbm, b_hbm, o_ref,
                  acc_ref, idx_ref, w_ref, b_ref, w_sem, b_sem):
    i = pl.program_id(0)

    @pl.when(i == 0)
    def _init():
        acc_ref[...] = jnp.zeros_like(acc_ref)

    @pl.when(i < _NB)
    def _router():
        part = jnp.dot(pred_ref[...], wp_ref[...],
                       preferred_element_type=jnp.float32)
        acc_ref[...] += jnp.sum(part, axis=0, keepdims=True)

    @pl.when(i == _NB - 1)
    def _pick_expert():
        scores = acc_ref[...] + jnp.float32(T) * bp_ref[...]
        m = jnp.max(scores)
        lane = jax.lax.broadcasted_iota(jnp.int32, scores.shape, 1)
        idx = jnp.min(jnp.where(scores == m, lane, jnp.int32(2**30)))
        idx_ref[0] = idx
        for k in range(_WCHUNKS):
            pltpu.make_async_copy(
                w_hbm.at[idx, pl.ds(k * _KC, _KC)],
                w_ref.at[pl.ds(k * _KC, _KC)], w_sem).start()
        pltpu.make_async_copy(b_hbm.at[idx], b_ref, b_sem).start()

    @pl.when(i == _NB)
    def _wait_w():
        for k in range(_WCHUNKS):
            pltpu.make_async_copy(
                w_hbm.at[idx_ref[0], pl.ds(k * _KC, _KC)],
                w_ref.at[pl.ds(k * _KC, _KC)], w_sem).wait()
        pltpu.make_async_copy(b_hbm.at[idx_ref[0]], b_ref, b_sem).wait()

    @pl.when(i >= _NB)
    def _dispatch():
        o_ref[...] = (jnp.dot(x_ref[...], w_ref[...],
                              preferred_element_type=jnp.float32)
                      + b_ref[...])


def kernel(predicate, input, W_pred, b_pred, W_experts, b_experts):
    bp2 = b_pred.reshape(1, E)
    be3 = b_experts.reshape(E, 1, D)

    out = pl.pallas_call(
        _fused_kernel,
        grid=(_STEPS,),
        in_specs=[
            pl.BlockSpec((_BM, D), lambda i: (jnp.minimum(i, _NB - 1), 0)),
            pl.BlockSpec((D, E), lambda i: (0, 0)),
            pl.BlockSpec((1, E), lambda i: (0, 0)),
            pl.BlockSpec((_BM, D), lambda i: (jnp.maximum(i - _NB, 0), 0)),
            pl.BlockSpec(memory_space=pltpu.MemorySpace.HBM),
            pl.BlockSpec(memory_space=pltpu.MemorySpace.HBM),
        ],
        out_specs=pl.BlockSpec((_BM, D), lambda i: (jnp.maximum(i - _NB, 0), 0)),
        out_shape=jax.ShapeDtypeStruct((T, D), jnp.float32),
        scratch_shapes=[
            pltpu.VMEM((1, E), jnp.float32),
            pltpu.SMEM((1,), jnp.int32),
            pltpu.VMEM((D, D), jnp.float32),
            pltpu.VMEM((1, D), jnp.float32),
            pltpu.SemaphoreType.DMA,
            pltpu.SemaphoreType.DMA,
        ],
    )(predicate, W_pred, bp2, input, W_experts, be3)
    return out


# speedup vs baseline: 1.0393x; 1.0393x over previous
import jax
import jax.numpy as jnp
from jax.experimental import pallas as pl
from jax.experimental.pallas import tpu as pltpu

T = 4096
D = 2048
E = 8

_BM = 512
_NB = T // _BM
_STEPS = 2 * _NB
_WCHUNKS = 8
_KC = D // _WCHUNKS


def _fused_kernel(pred_ref, wp_ref, bp_ref, x_ref, w_hbm, b_hbm, o_ref,
                  acc_ref, idx_ref, w_ref, b_ref, w_sem, b_sem):
    i = pl.program_id(0)

    @pl.when(i == 0)
    def _init():
        acc_ref[...] = jnp.zeros_like(acc_ref)

    @pl.when(i < _NB)
    def _router():
        part = jnp.dot(pred_ref[...], wp_ref[...],
                       preferred_element_type=jnp.float32)
        acc_ref[...] += jnp.sum(part, axis=0, keepdims=True)

    @pl.when(i == _NB - 1)
    def _pick_expert():
        scores = acc_ref[...] + jnp.float32(T) * bp_ref[...]
        m = jnp.max(scores)
        lane = jax.lax.broadcasted_iota(jnp.int32, scores.shape, 1)
        idx = jnp.min(jnp.where(scores == m, lane, jnp.int32(2**30)))
        idx_ref[0] = idx
        for k in range(_WCHUNKS):
            pltpu.make_async_copy(
                w_hbm.at[idx, pl.ds(k * _KC, _KC)],
                w_ref.at[pl.ds(k * _KC, _KC)], w_sem.at[k]).start()
        pltpu.make_async_copy(b_hbm.at[idx], b_ref, b_sem).start()

    @pl.when(i == _NB)
    def _first_dispatch():
        pltpu.make_async_copy(b_hbm.at[idx_ref[0]], b_ref, b_sem).wait()
        for k in range(_WCHUNKS):
            pltpu.make_async_copy(
                w_hbm.at[idx_ref[0], pl.ds(k * _KC, _KC)],
                w_ref.at[pl.ds(k * _KC, _KC)], w_sem.at[k]).wait()
            part = jnp.dot(x_ref[:, k * _KC:(k + 1) * _KC],
                           w_ref[pl.ds(k * _KC, _KC), :],
                           preferred_element_type=jnp.float32)
            if k == 0:
                o_ref[...] = part + b_ref[...]
            else:
                o_ref[...] += part

    @pl.when(i > _NB)
    def _dispatch():
        o_ref[...] = (jnp.dot(x_ref[...], w_ref[...],
                              preferred_element_type=jnp.float32)
                      + b_ref[...])


def kernel(predicate, input, W_pred, b_pred, W_experts, b_experts):
    bp2 = b_pred.reshape(1, E)
    be3 = b_experts.reshape(E, 1, D)

    out = pl.pallas_call(
        _fused_kernel,
        grid=(_STEPS,),
        in_specs=[
            pl.BlockSpec((_BM, D), lambda i: (jnp.minimum(i, _NB - 1), 0)),
            pl.BlockSpec((D, E), lambda i: (0, 0)),
            pl.BlockSpec((1, E), lambda i: (0, 0)),
            pl.BlockSpec((_BM, D), lambda i: (jnp.maximum(i - _NB, 0), 0)),
            pl.BlockSpec(memory_space=pltpu.MemorySpace.HBM),
            pl.BlockSpec(memory_space=pltpu.MemorySpace.HBM),
        ],
        out_specs=pl.BlockSpec((_BM, D), lambda i: (jnp.maximum(i - _NB, 0), 0)),
        out_shape=jax.ShapeDtypeStruct((T, D), jnp.float32),
        scratch_shapes=[
            pltpu.VMEM((1, E), jnp.float32),
            pltpu.SMEM((1,), jnp.int32),
            pltpu.VMEM((D, D), jnp.float32),
            pltpu.VMEM((1, D), jnp.float32),
            pltpu.SemaphoreType.DMA((_WCHUNKS,)),
            pltpu.SemaphoreType.DMA,
        ],
    )(predicate, W_pred, bp2, input, W_experts, be3)
    return out


# speedup vs baseline: 1.0467x; 1.0071x over previous
import jax
import jax.numpy as jnp
from jax.experimental import pallas as pl
from jax.experimental.pallas import tpu as pltpu

T = 4096
D = 2048
E = 8

_BM = 512
_NB = T // _BM
_NR = _NB // 2
_STEPS = _NR + _NB
_WCHUNKS = 8
_KC = D // _WCHUNKS


def _fused_kernel(pa_ref, pb_ref, wp_ref, bp_ref, x_ref, w_hbm, b_hbm, o_ref,
                  acc_ref, idx_ref, w_ref, b_ref, w_sem, b_sem):
    i = pl.program_id(0)

    @pl.when(i == 0)
    def _init():
        acc_ref[...] = jnp.zeros_like(acc_ref)

    @pl.when(i < _NR)
    def _router():
        part_a = jnp.dot(pa_ref[...], wp_ref[...],
                         preferred_element_type=jnp.float32)
        part_b = jnp.dot(pb_ref[...], wp_ref[...],
                         preferred_element_type=jnp.float32)
        acc_ref[...] += jnp.sum(part_a + part_b, axis=0, keepdims=True)

    @pl.when(i == _NR - 1)
    def _pick_expert():
        scores = acc_ref[...] + jnp.float32(T) * bp_ref[...]
        m = jnp.max(scores)
        lane = jax.lax.broadcasted_iota(jnp.int32, scores.shape, 1)
        idx = jnp.min(jnp.where(scores == m, lane, jnp.int32(2**30)))
        idx_ref[0] = idx
        for k in range(_WCHUNKS):
            pltpu.make_async_copy(
                w_hbm.at[idx, pl.ds(k * _KC, _KC)],
                w_ref.at[pl.ds(k * _KC, _KC)], w_sem.at[k]).start()
        pltpu.make_async_copy(b_hbm.at[idx], b_ref, b_sem).start()

    @pl.when(i == _NR)
    def _first_dispatch():
        pltpu.make_async_copy(b_hbm.at[idx_ref[0]], b_ref, b_sem).wait()
        for k in range(_WCHUNKS):
            pltpu.make_async_copy(
                w_hbm.at[idx_ref[0], pl.ds(k * _KC, _KC)],
                w_ref.at[pl.ds(k * _KC, _KC)], w_sem.at[k]).wait()
            part = jnp.dot(x_ref[:, k * _KC:(k + 1) * _KC],
                           w_ref[pl.ds(k * _KC, _KC), :],
                           preferred_element_type=jnp.float32)
            if k == 0:
                o_ref[...] = part + b_ref[...]
            else:
                o_ref[...] += part

    @pl.when(i > _NR)
    def _dispatch():
        o_ref[...] = (jnp.dot(x_ref[...], w_ref[...],
                              preferred_element_type=jnp.float32)
                      + b_ref[...])


def kernel(predicate, input, W_pred, b_pred, W_experts, b_experts):
    bp2 = b_pred.reshape(1, E)
    be3 = b_experts.reshape(E, 1, D)

    out = pl.pallas_call(
        _fused_kernel,
        grid=(_STEPS,),
        in_specs=[
            pl.BlockSpec((_BM, D), lambda i: (jnp.minimum(i, _NR - 1), 0)),
            pl.BlockSpec((_BM, D),
                         lambda i: (jnp.minimum(i, _NR - 1) + _NR, 0)),
            pl.BlockSpec((D, E), lambda i: (0, 0)),
            pl.BlockSpec((1, E), lambda i: (0, 0)),
            pl.BlockSpec((_BM, D), lambda i: (jnp.maximum(i - _NR, 0), 0)),
            pl.BlockSpec(memory_space=pltpu.MemorySpace.HBM),
            pl.BlockSpec(memory_space=pltpu.MemorySpace.HBM),
        ],
        out_specs=pl.BlockSpec((_BM, D), lambda i: (jnp.maximum(i - _NR, 0), 0)),
        out_shape=jax.ShapeDtypeStruct((T, D), jnp.float32),
        scratch_shapes=[
            pltpu.VMEM((1, E), jnp.float32),
            pltpu.SMEM((1,), jnp.int32),
            pltpu.VMEM((D, D), jnp.float32),
            pltpu.VMEM((1, D), jnp.float32),
            pltpu.SemaphoreType.DMA((_WCHUNKS,)),
            pltpu.SemaphoreType.DMA,
        ],
    )(predicate, predicate, W_pred, bp2, input, W_experts, be3)
    return out
